# baseline (speedup 1.0000x reference)
import functools

import jax
import jax.numpy as jnp
from jax import lax
from jax.experimental import pallas as pl
from jax.experimental.pallas import tpu as pltpu
from jax.experimental.pallas import tpu_sc as plsc

N = 10000
E = 320000
D = 128
NG = 64
NC = 2
NS = 16
NW = NC * NS
L = 128
S0 = 80
S1 = 80
SMAX = max(S0, S1)
TOT = NS * (S0 + S1)
STEPS = S0 + S1
EPAD = TOT * L
NPAD = 10240
TROWS = NPAD // NS
R = 1024
GR = NPAD // R

_f32 = jnp.float32
_mesh = plsc.VectorSubcoreMesh(
    core_axis_name="c", subcore_axis_name="s", num_cores=NC, num_subcores=NS)


def _hist_body(cols_hbm, zeros_hbm, ones_hbm, out,
               cols_v, ones_v, hist):
    cid = lax.axis_index("c")
    sid = lax.axis_index("s")
    wid = cid * NS + sid
    sl = pl.ds(sid * TROWS, TROWS)
    pltpu.sync_copy(zeros_hbm, hist.at[sl])
    pltpu.sync_copy(ones_hbm, ones_v)

    def step(j, c):
        pltpu.sync_copy(ones_v, hist.at[cols_v.at[j]], add=True)
        return c

    @pl.when(cid == 0)
    def _():
        pltpu.sync_copy(cols_hbm.at[wid, pl.ds(0, S0)], cols_v.at[pl.ds(0, S0)])
        plsc.subcore_barrier()
        lax.fori_loop(0, S0, step, 0)

    @pl.when(cid == 1)
    def _():
        pltpu.sync_copy(cols_hbm.at[wid, pl.ds(0, S1)], cols_v.at[pl.ds(0, S1)])
        plsc.subcore_barrier()
        lax.fori_loop(0, S1, step, 0)

    plsc.subcore_barrier()
    pltpu.sync_copy(hist.at[sl], out.at[pl.ds(cid * NPAD + sid * TROWS, TROWS)])


_hist_call = functools.partial(
    pl.kernel,
    out_type=jax.ShapeDtypeStruct((2 * NPAD, D), _f32),
    mesh=_mesh,
    scratch_types=[
        pltpu.VMEM((SMAX, L), jnp.int32),
        pltpu.VMEM((L, D), _f32),
        pltpu.VMEM_SHARED((NPAD, D), _f32),
    ],
)(_hist_body)


def _edge_body(y_hbm, rows_hbm, cols_hbm, zeros_hbm, out,
               rci, cci, buf, acc, sem):
    cid = lax.axis_index("c")
    sid = lax.axis_index("s")
    wid = cid * NS + sid
    sl = pl.ds(sid * TROWS, TROWS)
    pltpu.sync_copy(zeros_hbm, acc.at[sl])
    plsc.subcore_barrier()

    def step(j, c):
        pltpu.async_copy(y_hbm.at[rci.at[j]], buf, sem).wait()
        pltpu.sync_copy(buf, acc.at[cci.at[j]], add=True)
        return c

    @pl.when(cid == 0)
    def _():
        pltpu.sync_copy(rows_hbm.at[wid, pl.ds(0, S0)], rci.at[pl.ds(0, S0)])
        pltpu.sync_copy(cols_hbm.at[wid, pl.ds(0, S0)], cci.at[pl.ds(0, S0)])
        lax.fori_loop(0, S0, step, 0)

    @pl.when(cid == 1)
    def _():
        pltpu.sync_copy(rows_hbm.at[wid, pl.ds(0, S1)], rci.at[pl.ds(0, S1)])
        pltpu.sync_copy(cols_hbm.at[wid, pl.ds(0, S1)], cci.at[pl.ds(0, S1)])
        lax.fori_loop(0, S1, step, 0)

    plsc.subcore_barrier()
    pltpu.sync_copy(acc.at[sl], out.at[pl.ds(cid * NPAD + sid * TROWS, TROWS)])


_edge_call = functools.partial(
    pl.kernel,
    out_type=jax.ShapeDtypeStruct((2 * NPAD, D), _f32),
    mesh=_mesh,
    scratch_types=[
        pltpu.VMEM((SMAX, L), jnp.int32),
        pltpu.VMEM((SMAX, L), jnp.int32),
        pltpu.VMEM((L, D), _f32),
        pltpu.VMEM_SHARED((NPAD, D), _f32),
        pltpu.SemaphoreType.DMA,
    ],
)(_edge_body)


def _mm_nt(a, b):
    return lax.dot_general(a, b, (((1,), (1,)), ((), ())),
                           precision=lax.Precision.DEFAULT,
                           preferred_element_type=_f32)


def _mm_nn(a, b):
    return lax.dot_general(a, b, (((1,), (0,)), ((), ())),
                           precision=lax.Precision.HIGHEST,
                           preferred_element_type=_f32)


def _pre_body(x_ref, w_ref, h0_ref, h1_ref, y_ref, dinv_ref):
    i = pl.program_id(0)
    deg = h0_ref[:, 0:1] + h1_ref[:, 0:1] + 2.0
    rid = i * R + lax.broadcasted_iota(jnp.int32, (R, 1), 0)
    dinv = jnp.where(rid < N, lax.rsqrt(deg), 0.0)
    dinv_ref[...] = dinv
    y_ref[...] = dinv * _mm_nt(x_ref[...], w_ref[...])


def _mid_body(a0_ref, a1_ref, y_ref, dinv_ref, b_ref, w_ref, yo_ref):
    dinv = dinv_ref[...]
    xn = jnp.maximum(
        dinv * (a0_ref[...] + a1_ref[...] + 2.0 * y_ref[...]) + b_ref[...],
        0.0)
    yo_ref[...] = dinv * _mm_nt(xn, w_ref[...])


def _final_body(a0_ref, a1_ref, y_ref, dinv_ref, b_ref, batch_ref,
                fc1w_ref, fc1b_ref, fc2w_ref, fc2b_ref, out_ref,
                sums, cnt):
    i = pl.program_id(0)

    @pl.when(i == 0)
    def _():
        sums[...] = jnp.zeros((NG, D), _f32)
        cnt[...] = jnp.zeros((NG, 1), _f32)

    dinv = dinv_ref[...]
    x3 = jnp.maximum(
        dinv * (a0_ref[...] + a1_ref[...] + 2.0 * y_ref[...]) + b_ref[...],
        0.0)
    gids = lax.broadcasted_iota(jnp.int32, (NG, 1), 0)
    mask = (batch_ref[...] == gids).astype(_f32)
    sums[...] += _mm_nn(mask, x3)
    cnt[...] += jnp.sum(mask, axis=1, keepdims=True)

    @pl.when(i == GR - 1)
    def _():
        pooled = sums[...] / jnp.maximum(cnt[...], 1.0)
        h = jnp.maximum(_mm_nt(pooled, fc1w_ref[...]) + fc1b_ref[...], 0.0)
        hb = h.astype(jnp.bfloat16).astype(_f32)
        wb = fc2w_ref[...].astype(jnp.bfloat16).astype(_f32)
        out_ref[...] = (jnp.sum(hb * wb, axis=1, keepdims=True)
                        + fc2b_ref[0, 0])


_row_spec = pl.BlockSpec((R, D), lambda i: (i, 0))
_a1_spec = pl.BlockSpec((R, D), lambda i: (GR + i, 0))
_w_spec = pl.BlockSpec((D, D), lambda i: (0, 0))
_dinv_spec = pl.BlockSpec((R, 1), lambda i: (i, 0))
_b_spec = pl.BlockSpec((1, D), lambda i: (0, 0))

_pre_call = pl.pallas_call(
    _pre_body,
    grid=(GR,),
    in_specs=[_row_spec, _w_spec, _row_spec, _a1_spec],
    out_specs=(_row_spec, _dinv_spec),
    out_shape=(jax.ShapeDtypeStruct((NPAD, D), _f32),
               jax.ShapeDtypeStruct((NPAD, 1), _f32)),
)

_mid_call = pl.pallas_call(
    _mid_body,
    grid=(GR,),
    in_specs=[_row_spec, _a1_spec, _row_spec, _dinv_spec, _b_spec, _w_spec],
    out_specs=_row_spec,
    out_shape=jax.ShapeDtypeStruct((NPAD, D), _f32),
)

_final_call = pl.pallas_call(
    _final_body,
    grid=(GR,),
    in_specs=[_row_spec, _a1_spec, _row_spec, _dinv_spec, _b_spec,
              pl.BlockSpec((1, R), lambda i: (0, i)),
              _w_spec, _b_spec,
              pl.BlockSpec((1, D), lambda i: (0, 0)),
              pl.BlockSpec((1, 1), lambda i: (0, 0))],
    out_specs=pl.BlockSpec((NG, 1), lambda i: (0, 0)),
    out_shape=jax.ShapeDtypeStruct((NG, 1), _f32),
    scratch_shapes=[pltpu.VMEM((NG, D), _f32), pltpu.VMEM((NG, 1), _f32)],
)


def kernel(x, edge_index, batch, batch_size, W0, b0, W1, b1, W2, b2,
           fc1_W, fc1_b, fc2_W, fc2_b):
    x_pad = jnp.pad(x, ((0, NPAD - N), (0, 0)))
    dummy = jnp.full((EPAD - E,), NPAD - 1, jnp.int32)
    cap0 = NS * S0 * L

    def to_tiles(v):
        flat = jnp.concatenate([v, dummy])
        p0 = flat[:cap0].reshape(NS, S0, L)
        p1 = flat[cap0:].reshape(NS, S1, L)
        p0 = jnp.pad(p0, ((0, 0), (0, SMAX - S0), (0, 0)))
        p1 = jnp.pad(p1, ((0, 0), (0, SMAX - S1), (0, 0)))
        return jnp.concatenate([p0, p1], axis=0)

    rows = to_tiles(edge_index[0])
    cols = to_tiles(edge_index[1])
    batch_row = jnp.pad(batch, (0, NPAD - N),
                        constant_values=NG).reshape(1, NPAD)
    onesD = jnp.ones((L, D), _f32)
    zerosD = jnp.zeros((TROWS, D), _f32)

    hh = _hist_call(cols, zerosD, onesD)
    y, dinv = _pre_call(x_pad, W0, hh, hh)
    aa = _edge_call(y, rows, cols, zerosD)
    y = _mid_call(aa, aa, y, dinv, b0.reshape(1, D), W1)
    aa = _edge_call(y, rows, cols, zerosD)
    y = _mid_call(aa, aa, y, dinv, b1.reshape(1, D), W2)
    aa = _edge_call(y, rows, cols, zerosD)
    out = _final_call(aa, aa, y, dinv, b2.reshape(1, D), batch_row,
                      fc1_W, fc1_b.reshape(1, D), fc2_W,
                      fc2_b.reshape(1, 1))
    return out[:, 0]


# speedup vs baseline: 2.5466x; 2.5466x over previous
import functools

import jax
import jax.numpy as jnp
from jax import lax
from jax.experimental import pallas as pl
from jax.experimental.pallas import tpu as pltpu
from jax.experimental.pallas import tpu_sc as plsc

N = 10000
E = 320000
D = 128
NG = 64
NC = 2
NS = 16
NW = NC * NS
L = 128
STEPS = (-(-E // (NW * L)) + 7) // 8 * 8
EPAD = NW * L * STEPS
NPAD = 10240
TROWS = NPAD // NS
R = 1024
GR = NPAD // R

_f32 = jnp.float32
_mesh = plsc.VectorSubcoreMesh(
    core_axis_name="c", subcore_axis_name="s", num_cores=NC, num_subcores=NS)


def _hist_body(cols_hbm, zeros_hbm, ones_hbm, out,
               cols_v, ones_v, hist):
    cid = lax.axis_index("c")
    sid = lax.axis_index("s")
    wid = cid * NS + sid
    sl = pl.ds(sid * TROWS, TROWS)
    pltpu.sync_copy(zeros_hbm, hist.at[sl])
    pltpu.sync_copy(cols_hbm.at[wid], cols_v)
    pltpu.sync_copy(ones_hbm, ones_v)
    plsc.subcore_barrier()

    def step(j, c):
        pltpu.sync_copy(ones_v, hist.at[cols_v.at[j]], add=True)
        return c
    lax.fori_loop(0, STEPS, step, 0)
    plsc.subcore_barrier()
    pltpu.sync_copy(hist.at[sl], out.at[pl.ds(cid * NPAD + sid * TROWS, TROWS)])


_hist_call = functools.partial(
    pl.kernel,
    out_type=jax.ShapeDtypeStruct((2 * NPAD, D), _f32),
    mesh=_mesh,
    scratch_types=[
        pltpu.VMEM((STEPS, L), jnp.int32),
        pltpu.VMEM((L, D), _f32),
        pltpu.VMEM_SHARED((NPAD, D), _f32),
    ],
)(_hist_body)


def _edge_body(y_hbm, rows_hbm, cols_hbm, zeros_hbm, out,
               rows_v, cols_v, buf, acc, sem):
    cid = lax.axis_index("c")
    sid = lax.axis_index("s")
    wid = cid * NS + sid
    sl = pl.ds(sid * TROWS, TROWS)
    pltpu.sync_copy(zeros_hbm, acc.at[sl])
    pltpu.sync_copy(rows_hbm.at[wid], rows_v)
    pltpu.sync_copy(cols_hbm.at[wid], cols_v)
    plsc.subcore_barrier()

    def step(j, c):
        pltpu.async_copy(y_hbm.at[rows_v.at[j]], buf, sem).wait()
        pltpu.sync_copy(buf, acc.at[cols_v.at[j]], add=True)
        return c
    lax.fori_loop(0, STEPS, step, 0)
    plsc.subcore_barrier()
    pltpu.sync_copy(acc.at[sl], out.at[pl.ds(cid * NPAD + sid * TROWS, TROWS)])


_edge_call = functools.partial(
    pl.kernel,
    out_type=jax.ShapeDtypeStruct((2 * NPAD, D), _f32),
    mesh=_mesh,
    scratch_types=[
        pltpu.VMEM((STEPS, L), jnp.int32),
        pltpu.VMEM((STEPS, L), jnp.int32),
        pltpu.VMEM((L, D), _f32),
        pltpu.VMEM_SHARED((NPAD, D), _f32),
        pltpu.SemaphoreType.DMA,
    ],
)(_edge_body)


def _mm_nt(a, b):
    return lax.dot_general(a, b, (((1,), (1,)), ((), ())),
                           precision=lax.Precision.DEFAULT,
                           preferred_element_type=_f32)


def _mm_nn(a, b):
    return lax.dot_general(a, b, (((1,), (0,)), ((), ())),
                           precision=lax.Precision.HIGHEST,
                           preferred_element_type=_f32)


def _pre_body(x_ref, w_ref, h0_ref, h1_ref, y_ref, dinv_ref):
    i = pl.program_id(0)
    deg = h0_ref[:, 0:1] + h1_ref[:, 0:1] + 2.0
    rid = i * R + lax.broadcasted_iota(jnp.int32, (R, 1), 0)
    dinv = jnp.where(rid < N, lax.rsqrt(deg), 0.0)
    dinv_ref[...] = dinv
    y_ref[...] = dinv * _mm_nt(x_ref[...], w_ref[...])


def _mid_body(a0_ref, a1_ref, y_ref, dinv_ref, b_ref, w_ref, yo_ref):
    dinv = dinv_ref[...]
    xn = jnp.maximum(
        dinv * (a0_ref[...] + a1_ref[...] + 2.0 * y_ref[...]) + b_ref[...],
        0.0)
    yo_ref[...] = dinv * _mm_nt(xn, w_ref[...])


def _final_body(a0_ref, a1_ref, y_ref, dinv_ref, b_ref, batch_ref,
                fc1w_ref, fc1b_ref, fc2w_ref, fc2b_ref, out_ref,
                sums, cnt):
    i = pl.program_id(0)

    @pl.when(i == 0)
    def _():
        sums[...] = jnp.zeros((NG, D), _f32)
        cnt[...] = jnp.zeros((NG, 1), _f32)

    dinv = dinv_ref[...]
    x3 = jnp.maximum(
        dinv * (a0_ref[...] + a1_ref[...] + 2.0 * y_ref[...]) + b_ref[...],
        0.0)
    gids = lax.broadcasted_iota(jnp.int32, (NG, 1), 0)
    mask = (batch_ref[...] == gids).astype(_f32)
    sums[...] += _mm_nn(mask, x3)
    cnt[...] += jnp.sum(mask, axis=1, keepdims=True)

    @pl.when(i == GR - 1)
    def _():
        pooled = sums[...] / jnp.maximum(cnt[...], 1.0)
        h = jnp.maximum(_mm_nt(pooled, fc1w_ref[...]) + fc1b_ref[...], 0.0)
        hb = h.astype(jnp.bfloat16).astype(_f32)
        wb = fc2w_ref[...].astype(jnp.bfloat16).astype(_f32)
        out_ref[...] = (jnp.sum(hb * wb, axis=1, keepdims=True)
                        + fc2b_ref[0, 0])


_row_spec = pl.BlockSpec((R, D), lambda i: (i, 0))
_a1_spec = pl.BlockSpec((R, D), lambda i: (GR + i, 0))
_w_spec = pl.BlockSpec((D, D), lambda i: (0, 0))
_dinv_spec = pl.BlockSpec((R, 1), lambda i: (i, 0))
_b_spec = pl.BlockSpec((1, D), lambda i: (0, 0))

_pre_call = pl.pallas_call(
    _pre_body,
    grid=(GR,),
    in_specs=[_row_spec, _w_spec, _row_spec, _a1_spec],
    out_specs=(_row_spec, _dinv_spec),
    out_shape=(jax.ShapeDtypeStruct((NPAD, D), _f32),
               jax.ShapeDtypeStruct((NPAD, 1), _f32)),
)

_mid_call = pl.pallas_call(
    _mid_body,
    grid=(GR,),
    in_specs=[_row_spec, _a1_spec, _row_spec, _dinv_spec, _b_spec, _w_spec],
    out_specs=_row_spec,
    out_shape=jax.ShapeDtypeStruct((NPAD, D), _f32),
)

_final_call = pl.pallas_call(
    _final_body,
    grid=(GR,),
    in_specs=[_row_spec, _a1_spec, _row_spec, _dinv_spec, _b_spec,
              pl.BlockSpec((1, R), lambda i: (0, i)),
              _w_spec, _b_spec,
              pl.BlockSpec((1, D), lambda i: (0, 0)),
              pl.BlockSpec((1, 1), lambda i: (0, 0))],
    out_specs=pl.BlockSpec((NG, 1), lambda i: (0, 0)),
    out_shape=jax.ShapeDtypeStruct((NG, 1), _f32),
    scratch_shapes=[pltpu.VMEM((NG, D), _f32), pltpu.VMEM((NG, 1), _f32)],
)


def kernel(x, edge_index, batch, batch_size, W0, b0, W1, b1, W2, b2,
           fc1_W, fc1_b, fc2_W, fc2_b):
    x_pad = jnp.pad(x, ((0, NPAD - N), (0, 0)))
    dummy = N + jnp.arange(EPAD - E, dtype=jnp.int32) % (NPAD - N)
    rows = jnp.concatenate([edge_index[0], dummy]).reshape(NW, STEPS, L)
    cols = jnp.concatenate([edge_index[1], dummy]).reshape(NW, STEPS, L)
    batch_row = jnp.pad(batch, (0, NPAD - N),
                        constant_values=NG).reshape(1, NPAD)
    onesD = jnp.ones((L, D), _f32)
    zerosD = jnp.zeros((TROWS, D), _f32)

    hh = _hist_call(cols, zerosD, onesD)
    y, dinv = _pre_call(x_pad, W0, hh, hh)
    aa = _edge_call(y, rows, cols, zerosD)
    y = _mid_call(aa, aa, y, dinv, b0.reshape(1, D), W1)
    aa = _edge_call(y, rows, cols, zerosD)
    y = _mid_call(aa, aa, y, dinv, b1.reshape(1, D), W2)
    aa = _edge_call(y, rows, cols, zerosD)
    out = _final_call(aa, aa, y, dinv, b2.reshape(1, D), batch_row,
                      fc1_W, fc1_b.reshape(1, D), fc2_W,
                      fc2_b.reshape(1, 1))
    return out[:, 0]
